# baseline (speedup 1.0000x reference)
import jax
import jax.numpy as jnp
from jax import lax
from jax.experimental import pallas as pl
from jax.experimental.pallas import tpu as pltpu
from jax.experimental.pallas import tpu_sc as plsc

B, C, N = 64, 16, 4096
SEG = 128
BLK = 256
NCG = N // BLK
NUM_CORES = 2
NUM_SUBCORES = 16
NW = NUM_CORES * NUM_SUBCORES
UNITS = B * NCG
UPW = UNITS // NW

A0 = (-1.0 / 6.0, 2.0 / 3.0, 2.0 / 3.0, -1.0 / 6.0)
A1 = (1.0 / 6.0, -4.0 / 3.0, 4.0 / 3.0, -1.0 / 6.0)
A2 = (2.0 / 3.0, -2.0 / 3.0, -2.0 / 3.0, 2.0 / 3.0)
A3 = (-2.0 / 3.0, 4.0 / 3.0, -4.0 / 3.0, 2.0 / 3.0)


def _sc_body(x_hbm, w_hbm, out_hbm,
             xid0, xid1, xb0, xb1, tab_v, tab2_v, o0, o1, si0, si1, so0, so1):
    wid = lax.axis_index("s") * NUM_CORES + lax.axis_index("c")
    base = wid * UPW
    pltpu.sync_copy(w_hbm, tab_v)
    iota = lax.iota(jnp.int32, 16)
    iota3 = iota * 3
    iota129 = iota * (SEG + 1)

    @plsc.parallel_loop(0, C * (SEG // 16), unroll=2)
    def fold_body(i):
        c = i // (SEG // 16)
        sk = i - c * (SEG // 16)
        cs = jnp.full((16,), c, dtype=jnp.int32)
        widx = iota3 + sk * 48
        w0 = plsc.load_gather(tab_v, [cs, widx])
        w1 = plsc.load_gather(tab_v, [cs, widx + 1])
        w2 = plsc.load_gather(tab_v, [cs, widx + 2])
        w3 = plsc.load_gather(tab_v, [cs, widx + 3])
        scol = cs * (SEG + 1) + iota + sk * 16
        for k, AK in enumerate((A0, A1, A2, A3)):
            p = AK[0] * w0 + AK[1] * w1 + AK[2] * w2 + AK[3] * w3
            plsc.store_scatter(tab2_v.at[k], [scol], p)

    def unit_bcg(u):
        unit = base + u
        b = unit // NCG
        cg = unit - b * NCG
        return b, cg, cg * BLK

    def start_in(u, xid_v, xb_v, sem):
        b, cg, col0 = unit_bcg(u)
        pltpu.async_copy(x_hbm.at[b, cg], xid_v, sem)
        pltpu.async_copy(x_hbm.at[b, :, pl.ds(col0, BLK)],
                         xb_v.at[:, pl.ds(0, BLK)], sem)

    def wait_in(xid_v, xb_v, sem):
        pltpu.make_async_copy(x_hbm.at[0, 0], xid_v, sem).wait()
        pltpu.make_async_copy(x_hbm.at[0, :, pl.ds(0, BLK)],
                              xb_v.at[:, pl.ds(0, BLK)], sem).wait()

    def start_out(u, out_v, sem):
        b, _cg, col0 = unit_bcg(u)
        pltpu.async_copy(out_v.at[:, pl.ds(0, BLK)],
                         out_hbm.at[b, :, pl.ds(col0, BLK)], sem)

    def wait_out(out_v, sem):
        pltpu.make_async_copy(out_v.at[:, pl.ds(0, BLK)],
                              out_hbm.at[0, :, pl.ds(0, BLK)], sem).wait()

    def compute(xid_v, xb_v, out_v):
        @plsc.parallel_loop(0, BLK, unroll=3)
        def col_body(ih):
            va = xid_v[pl.ds(ih * 16, 16)]
            tmpa = (va + 1.0) * 64.0
            ida = jnp.maximum(jnp.minimum(tmpa.astype(jnp.int32), SEG - 1), 0)
            gv = iota129 + ida
            p0 = plsc.load_gather(tab2_v.at[0], [gv])
            p1 = plsc.load_gather(tab2_v.at[1], [gv])
            p2 = plsc.load_gather(tab2_v.at[2], [gv])
            p3 = plsc.load_gather(tab2_v.at[3], [gv])
            ihv = jnp.full((16,), ih, dtype=jnp.int32)
            xc = plsc.load_gather(xb_v, [iota, ihv])
            tmpb = (xc + 1.0) * 64.0
            idb = jnp.maximum(jnp.minimum(tmpb.astype(jnp.int32), SEG - 1), 0)
            t = 2.0 * (tmpb - idb.astype(jnp.float32)) - 1.0
            acc = ((p3 * t + p2) * t + p1) * t + p0
            plsc.store_scatter(out_v, [iota, ihv], acc)

    start_in(0, xid0, xb0, si0)

    def g_body(g, carry):
        u0 = 2 * g
        start_in(u0 + 1, xid1, xb1, si1)
        wait_in(xid0, xb0, si0)

        @pl.when(g >= 1)
        def _w0():
            wait_out(o0, so0)

        compute(xid0, xb0, o0)
        start_out(u0, o0, so0)

        @pl.when(g < UPW // 2 - 1)
        def _s0():
            start_in(u0 + 2, xid0, xb0, si0)

        wait_in(xid1, xb1, si1)

        @pl.when(g >= 1)
        def _w1():
            wait_out(o1, so1)

        compute(xid1, xb1, o1)
        start_out(u0 + 1, o1, so1)
        return carry

    lax.fori_loop(0, UPW // 2, g_body, None)
    wait_out(o0, so0)
    wait_out(o1, so1)


@jax.jit
def kernel(x, w_table):
    mesh = plsc.VectorSubcoreMesh(
        core_axis_name="c", subcore_axis_name="s",
        num_cores=NUM_CORES, num_subcores=NUM_SUBCORES,
    )
    f = pl.kernel(
        _sc_body,
        out_type=jax.ShapeDtypeStruct((B, C, N), jnp.float32),
        mesh=mesh,
        scratch_types=[
            pltpu.VMEM((N,), jnp.float32),
            pltpu.VMEM((N,), jnp.float32),
            pltpu.VMEM((C, BLK + 1), jnp.float32),
            pltpu.VMEM((C, BLK + 1), jnp.float32),
            pltpu.VMEM((C, 3 * SEG + 1), jnp.float32),
            pltpu.VMEM((4, C * (SEG + 1)), jnp.float32),
            pltpu.VMEM((C, BLK + 1), jnp.float32),
            pltpu.VMEM((C, BLK + 1), jnp.float32),
            pltpu.SemaphoreType.DMA,
            pltpu.SemaphoreType.DMA,
            pltpu.SemaphoreType.DMA,
            pltpu.SemaphoreType.DMA,
        ],
        compiler_params=pltpu.CompilerParams(
            use_tc_tiling_on_sc=False, needs_layout_passes=False,
        ),
    )
    return f(x, w_table)


# speedup vs baseline: 1.0021x; 1.0021x over previous
import jax
import jax.numpy as jnp
from jax import lax
from jax.experimental import pallas as pl
from jax.experimental.pallas import tpu as pltpu
from jax.experimental.pallas import tpu_sc as plsc

B, C, N = 64, 16, 4096
SEG = 128
BLK = 256
NCG = N // BLK
NUM_CORES = 2
NUM_SUBCORES = 16
NW = NUM_CORES * NUM_SUBCORES
UNITS = B * NCG
UPW = UNITS // NW

A0 = (-1.0 / 6.0, 2.0 / 3.0, 2.0 / 3.0, -1.0 / 6.0)
A1 = (1.0 / 6.0, -4.0 / 3.0, 4.0 / 3.0, -1.0 / 6.0)
A2 = (2.0 / 3.0, -2.0 / 3.0, -2.0 / 3.0, 2.0 / 3.0)
A3 = (-2.0 / 3.0, 4.0 / 3.0, -4.0 / 3.0, 2.0 / 3.0)


def _sc_body(x_hbm, w_hbm, out_hbm,
             xid0, xid1, xb0, xb1, tab_v, tab2_v, o0, o1, si0, si1, so0, so1):
    wid = lax.axis_index("s") * NUM_CORES + lax.axis_index("c")
    base = wid
    pltpu.sync_copy(w_hbm, tab_v)
    iota = lax.iota(jnp.int32, 16)
    iota3 = iota * 3
    iota129 = iota * (SEG + 1)

    @plsc.parallel_loop(0, C * (SEG // 16), unroll=2)
    def fold_body(i):
        c = i // (SEG // 16)
        sk = i - c * (SEG // 16)
        cs = jnp.full((16,), c, dtype=jnp.int32)
        widx = iota3 + sk * 48
        w0 = plsc.load_gather(tab_v, [cs, widx])
        w1 = plsc.load_gather(tab_v, [cs, widx + 1])
        w2 = plsc.load_gather(tab_v, [cs, widx + 2])
        w3 = plsc.load_gather(tab_v, [cs, widx + 3])
        scol = cs * (SEG + 1) + iota + sk * 16
        for k, AK in enumerate((A0, A1, A2, A3)):
            p = AK[0] * w0 + AK[1] * w1 + AK[2] * w2 + AK[3] * w3
            plsc.store_scatter(tab2_v.at[k], [scol], p)

    def unit_bcg(u):
        unit = u * NW + base
        b = unit // NCG
        cg = unit - b * NCG
        return b, cg, cg * BLK

    def start_in(u, xid_v, xb_v, sem):
        b, cg, col0 = unit_bcg(u)
        pltpu.async_copy(x_hbm.at[b, cg], xid_v, sem)
        pltpu.async_copy(x_hbm.at[b, :, pl.ds(col0, BLK)],
                         xb_v.at[:, pl.ds(0, BLK)], sem)

    def wait_in(xid_v, xb_v, sem):
        pltpu.make_async_copy(x_hbm.at[0, 0], xid_v, sem).wait()
        pltpu.make_async_copy(x_hbm.at[0, :, pl.ds(0, BLK)],
                              xb_v.at[:, pl.ds(0, BLK)], sem).wait()

    def start_out(u, out_v, sem):
        b, _cg, col0 = unit_bcg(u)
        pltpu.async_copy(out_v.at[:, pl.ds(0, BLK)],
                         out_hbm.at[b, :, pl.ds(col0, BLK)], sem)

    def wait_out(out_v, sem):
        pltpu.make_async_copy(out_v.at[:, pl.ds(0, BLK)],
                              out_hbm.at[0, :, pl.ds(0, BLK)], sem).wait()

    def compute(xid_v, xb_v, out_v):
        @plsc.parallel_loop(0, BLK, unroll=2)
        def col_body(ih):
            va = xid_v[pl.ds(ih * 16, 16)]
            tmpa = (va + 1.0) * 64.0
            ida = jnp.maximum(jnp.minimum(tmpa.astype(jnp.int32), SEG - 1), 0)
            gv = iota129 + ida
            p0 = plsc.load_gather(tab2_v.at[0], [gv])
            p1 = plsc.load_gather(tab2_v.at[1], [gv])
            p2 = plsc.load_gather(tab2_v.at[2], [gv])
            p3 = plsc.load_gather(tab2_v.at[3], [gv])
            ihv = jnp.full((16,), ih, dtype=jnp.int32)
            xc = plsc.load_gather(xb_v, [iota, ihv])
            tmpb = (xc + 1.0) * 64.0
            idb = jnp.maximum(jnp.minimum(tmpb.astype(jnp.int32), SEG - 1), 0)
            t = 2.0 * (tmpb - idb.astype(jnp.float32)) - 1.0
            acc = ((p3 * t + p2) * t + p1) * t + p0
            plsc.store_scatter(out_v, [iota, ihv], acc)

    start_in(0, xid0, xb0, si0)

    def g_body(g, carry):
        u0 = 2 * g
        start_in(u0 + 1, xid1, xb1, si1)
        wait_in(xid0, xb0, si0)

        @pl.when(g >= 1)
        def _w0():
            wait_out(o0, so0)

        compute(xid0, xb0, o0)
        start_out(u0, o0, so0)

        @pl.when(g < UPW // 2 - 1)
        def _s0():
            start_in(u0 + 2, xid0, xb0, si0)

        wait_in(xid1, xb1, si1)

        @pl.when(g >= 1)
        def _w1():
            wait_out(o1, so1)

        compute(xid1, xb1, o1)
        start_out(u0 + 1, o1, so1)
        return carry

    lax.fori_loop(0, UPW // 2, g_body, None)
    wait_out(o0, so0)
    wait_out(o1, so1)


@jax.jit
def kernel(x, w_table):
    mesh = plsc.VectorSubcoreMesh(
        core_axis_name="c", subcore_axis_name="s",
        num_cores=NUM_CORES, num_subcores=NUM_SUBCORES,
    )
    f = pl.kernel(
        _sc_body,
        out_type=jax.ShapeDtypeStruct((B, C, N), jnp.float32),
        mesh=mesh,
        scratch_types=[
            pltpu.VMEM((N,), jnp.float32),
            pltpu.VMEM((N,), jnp.float32),
            pltpu.VMEM((C, BLK + 1), jnp.float32),
            pltpu.VMEM((C, BLK + 1), jnp.float32),
            pltpu.VMEM((C, 3 * SEG + 1), jnp.float32),
            pltpu.VMEM((4, C * (SEG + 1)), jnp.float32),
            pltpu.VMEM((C, BLK + 1), jnp.float32),
            pltpu.VMEM((C, BLK + 1), jnp.float32),
            pltpu.SemaphoreType.DMA,
            pltpu.SemaphoreType.DMA,
            pltpu.SemaphoreType.DMA,
            pltpu.SemaphoreType.DMA,
        ],
        compiler_params=pltpu.CompilerParams(
            use_tc_tiling_on_sc=False, needs_layout_passes=False,
        ),
    )
    return f(x, w_table)
